# baseline (speedup 1.0000x reference)
import functools

import jax
import jax.numpy as jnp
from jax import lax
from jax.experimental import pallas as pl
from jax.experimental.pallas import tpu as pltpu
from jax.experimental.pallas import tpu_sc as plsc

_BS = 4
_Q = 300
_P = 64
_N = _BS * _P
_NCH = 19
_SEG = 384
_NSEG = 5
_W = _SEG * _NSEG
_TOPK = 100
_MS = 112
_KEEP = 20
_THR = 0.7
_PPW = 8


def _prep_body(lg_ref, cx_ref, cy_ref, w_ref, h_ref, sx_ref, sy_ref, out_ref):
    lanes = lax.broadcasted_iota(jnp.int32, (_N, _SEG), 1)
    pad = ((0, 0), (0, _SEG - _Q))
    x = jnp.pad(lg_ref[...], pad)
    prob = 0.5 * (jnp.tanh(0.5 * x) + 1.0)
    out_ref[:, 0:_SEG] = jnp.where(lanes < _Q, prob, -1.0)
    sx = sx_ref[:, 0:1]
    sy = sy_ref[:, 0:1]
    cx = jnp.pad(cx_ref[...], pad)
    cy = jnp.pad(cy_ref[...], pad)
    bw = jnp.pad(w_ref[...], pad)
    bh = jnp.pad(h_ref[...], pad)
    out_ref[:, _SEG:2 * _SEG] = (cx - 0.5 * bw) * sx
    out_ref[:, 2 * _SEG:3 * _SEG] = (cy - 0.5 * bh) * sy
    out_ref[:, 3 * _SEG:4 * _SEG] = (cx + 0.5 * bw) * sx
    out_ref[:, 4 * _SEG:5 * _SEG] = (cy + 0.5 * bh) * sy


def _sc_body(buf_h, mi_h, of_h, oi_h,
             bufv, miv, ssv, idxv, sx1, sy1, sx2, sy2, supv, stf, sti):
    i32 = jnp.int32
    f32 = jnp.float32
    cid = lax.axis_index("c")
    sid = lax.axis_index("s")
    wid = sid * 2 + cid
    base_row = wid * _PPW
    pltpu.sync_copy(buf_h.at[pl.ds(base_row, _PPW)], bufv)
    pltpu.sync_copy(mi_h.at[pl.ds(base_row, _PPW)], miv)
    iota = lax.broadcasted_iota(i32, (16,), 0)
    zf = jnp.zeros((16,), f32)
    zi = jnp.zeros((16,), i32)
    m0 = iota == 0
    gdn = lax.GatherDimensionNumbers(offset_dims=(), collapsed_slice_dims=(0,),
                                     start_index_map=(0,))

    def _take(x, idx):
        return lax.gather(x, idx[:, None], dimension_numbers=gdn,
                          slice_sizes=(1,),
                          mode=lax.GatherScatterMode.PROMISE_IN_BOUNDS)

    perms = [iota ^ d for d in (1, 2, 4, 8)]

    def _allmax(x):
        for pm in perms:
            x = jnp.maximum(x, _take(x, pm))
        return x

    def _allmin(x):
        for pm in perms:
            x = jnp.minimum(x, _take(x, pm))
        return x

    shift_idx = [(jnp.maximum(iota - d, 0), iota >= d) for d in (1, 2, 4, 8)]

    def _cumsum(x):
        for sidx, smask in shift_idx:
            x = x + jnp.where(smask, _take(x, sidx), 0)
        return x

    lane15 = jnp.full((16,), 15, i32)
    big = jnp.full((16,), 512, i32)
    neg2 = jnp.full((16,), -2.0, f32)

    for tp in range(_PPW // 2):
        ta, tb = 2 * tp, 2 * tp + 1
        ta16 = jnp.full((16,), ta, i32)
        tb16 = jnp.full((16,), tb, i32)

        cms = []
        for t, tv in ((ta, ta16), (tb, tb16)):
            cm0 = jnp.full((16,), -3.0, f32)
            cm1 = jnp.full((16,), -3.0, f32)
            for c in range(_NCH):
                nm = _allmax(bufv[t, pl.ds(c * 16, 16)])
                if c < 16:
                    cm0 = jnp.where(iota == c, nm, cm0)
                else:
                    cm1 = jnp.where(iota == c - 16, nm, cm1)
            idxv[t, pl.ds(96, 16)] = zi
            ssv[t, pl.ds(96, 16)] = zf
            cms += [cm0, cm1]

        def pop1(r16, c0, c1, t16):
            mx = _allmax(jnp.maximum(c0, c1))
            cand = jnp.minimum(jnp.where(c0 == mx, iota, big),
                               jnp.where(c1 == mx, iota + 16, big))
            cb = _allmin(cand)
            cb16 = cb * 16
            chunk = plsc.load_gather(bufv, [t16, cb16 + iota])
            lbest = _allmin(jnp.where(chunk == mx, iota, big))
            gidx = cb16 + lbest
            plsc.store_scatter(ssv, [t16, r16], mx, mask=m0)
            plsc.store_scatter(idxv, [t16, r16], gidx, mask=m0)
            plsc.store_scatter(bufv, [t16, gidx], neg2, mask=m0)
            chunk2 = jnp.where(iota == lbest, -2.0, chunk)
            nm = _allmax(chunk2)
            c0 = jnp.where(iota == cb, nm, c0)
            c1 = jnp.where(iota == cb - 16, nm, c1)
            return c0, c1

        def pop(r, carry):
            a0, a1, b0, b1 = carry
            r16 = jnp.broadcast_to(r, (16,))
            a0, a1 = pop1(r16, a0, a1, ta16)
            b0, b1 = pop1(r16, b0, b1, tb16)
            return a0, a1, b0, b1

        lax.fori_loop(0, _TOPK, pop, tuple(cms))

        for t, tv in ((ta, ta16), (tb, tb16)):
            for c in range(7):
                idx_c = idxv[t, pl.ds(c * 16, 16)]
                sx1[t, pl.ds(c * 16, 16)] = plsc.load_gather(bufv, [tv, idx_c + _SEG])
                sy1[t, pl.ds(c * 16, 16)] = plsc.load_gather(bufv, [tv, idx_c + 2 * _SEG])
                sx2[t, pl.ds(c * 16, 16)] = plsc.load_gather(bufv, [tv, idx_c + 3 * _SEG])
                sy2[t, pl.ds(c * 16, 16)] = plsc.load_gather(bufv, [tv, idx_c + 4 * _SEG])
                supv[t, pl.ds(c * 16, 16)] = jnp.where(iota + c * 16 < _TOPK, 0, 1)

        for blk in range(4):
            c_lo = 2 * blk

            def nms1(i, i16, t, t16):
                xi1 = plsc.load_gather(sx1, [t16, i16])
                xi2 = plsc.load_gather(sy1, [t16, i16])
                xi3 = plsc.load_gather(sx2, [t16, i16])
                xi4 = plsc.load_gather(sy2, [t16, i16])
                ai = (xi3 - xi1) * (xi4 - xi2)
                act = plsc.load_gather(supv, [t16, i16]) == 0
                for c in range(c_lo, 7):
                    b1 = sx1[t, pl.ds(c * 16, 16)]
                    b2 = sy1[t, pl.ds(c * 16, 16)]
                    b3 = sx2[t, pl.ds(c * 16, 16)]
                    b4 = sy2[t, pl.ds(c * 16, 16)]
                    av = (b3 - b1) * (b4 - b2)
                    iw = jnp.maximum(jnp.minimum(b3, xi3) - jnp.maximum(b1, xi1), 0.0)
                    ih = jnp.maximum(jnp.minimum(b4, xi4) - jnp.maximum(b2, xi2), 0.0)
                    inter = iw * ih
                    iou = inter / jnp.maximum(av + ai - inter, 1e-9)
                    sold = supv[t, pl.ds(c * 16, 16)]
                    cond = act & (iou > _THR) & (iota + c * 16 > i)
                    supv[t, pl.ds(c * 16, 16)] = jnp.where(cond, 1, sold)

            def nms(i, carry):
                i16 = jnp.broadcast_to(i, (16,))
                nms1(i, i16, ta, ta16)
                nms1(i, i16, tb, tb16)
                return carry

            lax.fori_loop(32 * blk, min(32 * blk + 32, _TOPK), nms, 0)

    for t in range(_PPW):
        t16 = jnp.full((16,), t, i32)
        for a in range(_NSEG):
            stf[t, pl.ds(a * 32, 16)] = zf
            stf[t, pl.ds(a * 32 + 16, 16)] = zf
        base = zi
        for c in range(7):
            keep = supv[t, pl.ds(c * 16, 16)] == 0
            ki = keep.astype(i32)
            cum = _cumsum(ki)
            slot = base + cum - ki
            msel = keep & (slot < _KEEP)
            plsc.store_scatter(stf, [t16, slot], ssv[t, pl.ds(c * 16, 16)], mask=msel)
            plsc.store_scatter(stf, [t16, slot + 32], sx1[t, pl.ds(c * 16, 16)], mask=msel)
            plsc.store_scatter(stf, [t16, slot + 64], sy1[t, pl.ds(c * 16, 16)], mask=msel)
            plsc.store_scatter(stf, [t16, slot + 96], sx2[t, pl.ds(c * 16, 16)], mask=msel)
            plsc.store_scatter(stf, [t16, slot + 128], sy2[t, pl.ds(c * 16, 16)], mask=msel)
            base = base + _take(cum, lane15)
        mrow = miv[t, pl.ds(0, 16)]
        name = _allmax(jnp.where(iota == 0, mrow, -1))
        strt = _allmax(jnp.where(iota == 1, mrow, -1))
        endv = _allmax(jnp.where(iota == 2, mrow, -1))
        neg = jnp.full((16,), -1, i32)
        v0 = iota < base
        v1 = (iota + 16) < base
        sti[t, pl.ds(0, 16)] = jnp.where(v0, name, neg)
        sti[t, pl.ds(16, 16)] = jnp.where(v1, name, neg)
        sti[t, pl.ds(32, 16)] = jnp.where(v0, strt, neg)
        sti[t, pl.ds(48, 16)] = jnp.where(v1, strt, neg)
        sti[t, pl.ds(64, 16)] = jnp.where(v0, endv, neg)
        sti[t, pl.ds(80, 16)] = jnp.where(v1, endv, neg)

    pltpu.sync_copy(stf, of_h.at[pl.ds(base_row, _PPW)])
    pltpu.sync_copy(sti, oi_h.at[pl.ds(base_row, _PPW)])


def kernel(pred_logits, pred_boxes, target_sizes, pred_names, mask_infos):
    f32 = jnp.float32
    i32 = jnp.int32
    lg = pred_logits[:, 0, :, 1].reshape(_N, _Q)
    bx = pred_boxes[:, 0].reshape(_N, _Q, 4)
    img_w = jnp.repeat(target_sizes[:, 1], _P)
    img_h = jnp.repeat(target_sizes[:, 0], _P)
    sxf = jnp.broadcast_to(img_w[:, None], (_N, 128))
    syf = jnp.broadcast_to(img_h[:, None], (_N, 128))
    mi = jnp.pad(
        jnp.stack([pred_names.reshape(_N), mask_infos[..., 0].reshape(_N),
                   mask_infos[..., 1].reshape(_N)], axis=-1).astype(i32),
        ((0, 0), (0, 13)))

    buf = pl.pallas_call(
        _prep_body,
        out_shape=jax.ShapeDtypeStruct((_N, _W), f32),
    )(lg, bx[..., 0], bx[..., 1], bx[..., 2], bx[..., 3], sxf, syf)

    mesh = plsc.VectorSubcoreMesh(core_axis_name="c", subcore_axis_name="s",
                                  num_cores=2, num_subcores=16)
    sc = functools.partial(
        pl.kernel, mesh=mesh,
        compiler_params=pltpu.CompilerParams(needs_layout_passes=False),
        out_type=[jax.ShapeDtypeStruct((_N, 32 * _NSEG), f32),
                  jax.ShapeDtypeStruct((_N, 96), i32)],
        scratch_types=[pltpu.VMEM((_PPW, _W), f32), pltpu.VMEM((_PPW, 16), i32),
                       pltpu.VMEM((_PPW, _MS), f32), pltpu.VMEM((_PPW, _MS), i32)]
        + [pltpu.VMEM((_PPW, _MS), f32)] * 4
        + [pltpu.VMEM((_PPW, _MS), i32)]
        + [pltpu.VMEM((_PPW, 32 * _NSEG), f32), pltpu.VMEM((_PPW, 96), i32)],
    )(_sc_body)

    of, oi = sc(buf, mi)
    scores = of[:, 0:_KEEP].reshape(_BS, _P * _KEEP)
    boxes = jnp.stack([of[:, 32:32 + _KEEP], of[:, 64:64 + _KEEP],
                       of[:, 96:96 + _KEEP], of[:, 128:128 + _KEEP]],
                      axis=-1).reshape(_BS, _P * _KEEP, 4)
    names_o = oi[:, 0:_KEEP].reshape(_BS, _P * _KEEP)
    starts_o = oi[:, 32:32 + _KEEP].reshape(_BS, _P * _KEEP)
    ends_o = oi[:, 64:64 + _KEEP].reshape(_BS, _P * _KEEP)
    return scores, boxes, names_o, starts_o, ends_o


# speedup vs baseline: 1.0436x; 1.0436x over previous
import functools

import jax
import jax.numpy as jnp
from jax import lax
from jax.experimental import pallas as pl
from jax.experimental.pallas import tpu as pltpu
from jax.experimental.pallas import tpu_sc as plsc

_BS = 4
_Q = 300
_P = 64
_N = _BS * _P
_NCH = 19
_SEG = 384
_NSEG = 5
_W = _SEG * _NSEG
_TOPK = 100
_MS = 112
_KEEP = 20
_THR = 0.7
_PPW = 8


def _prep_body(lg_ref, cx_ref, cy_ref, w_ref, h_ref, sx_ref, sy_ref, out_ref):
    lanes = lax.broadcasted_iota(jnp.int32, (_N, _SEG), 1)
    pad = ((0, 0), (0, _SEG - _Q))
    x = jnp.pad(lg_ref[...], pad)
    prob = 0.5 * (jnp.tanh(0.5 * x) + 1.0)
    out_ref[:, 0:_SEG] = jnp.where(lanes < _Q, prob, -1.0)
    sx = sx_ref[:, 0:1]
    sy = sy_ref[:, 0:1]
    cx = jnp.pad(cx_ref[...], pad)
    cy = jnp.pad(cy_ref[...], pad)
    bw = jnp.pad(w_ref[...], pad)
    bh = jnp.pad(h_ref[...], pad)
    out_ref[:, _SEG:2 * _SEG] = (cx - 0.5 * bw) * sx
    out_ref[:, 2 * _SEG:3 * _SEG] = (cy - 0.5 * bh) * sy
    out_ref[:, 3 * _SEG:4 * _SEG] = (cx + 0.5 * bw) * sx
    out_ref[:, 4 * _SEG:5 * _SEG] = (cy + 0.5 * bh) * sy


def _sc_body(buf_h, mi_h, of_h, oi_h,
             bufv, miv, ssv, idxv, sx1, sy1, sx2, sy2, supv, stf, sti):
    i32 = jnp.int32
    f32 = jnp.float32
    cid = lax.axis_index("c")
    sid = lax.axis_index("s")
    wid = sid * 2 + cid
    base_row = wid * _PPW
    pltpu.sync_copy(buf_h.at[pl.ds(base_row, _PPW)], bufv)
    pltpu.sync_copy(mi_h.at[pl.ds(base_row, _PPW)], miv)
    iota = lax.broadcasted_iota(i32, (16,), 0)
    zf = jnp.zeros((16,), f32)
    zi = jnp.zeros((16,), i32)
    m0 = iota == 0
    gdn = lax.GatherDimensionNumbers(offset_dims=(), collapsed_slice_dims=(0,),
                                     start_index_map=(0,))

    def _take(x, idx):
        return lax.gather(x, idx[:, None], dimension_numbers=gdn,
                          slice_sizes=(1,),
                          mode=lax.GatherScatterMode.PROMISE_IN_BOUNDS)

    perms = [iota ^ d for d in (1, 2, 4, 8)]

    def _allmax(x):
        for pm in perms:
            x = jnp.maximum(x, _take(x, pm))
        return x

    def _allmin(x):
        for pm in perms:
            x = jnp.minimum(x, _take(x, pm))
        return x

    shift_idx = [(jnp.maximum(iota - d, 0), iota >= d) for d in (1, 2, 4, 8)]

    def _cumsum(x):
        for sidx, smask in shift_idx:
            x = x + jnp.where(smask, _take(x, sidx), 0)
        return x

    lane15 = jnp.full((16,), 15, i32)
    big = jnp.full((16,), 512, i32)
    neg2 = jnp.full((16,), -2.0, f32)

    _G = 4
    for tp in range(_PPW // _G):
        ts = [(_G * tp + j, jnp.full((16,), _G * tp + j, i32)) for j in range(_G)]

        cms = []
        for t, tv in ts:
            cm0 = jnp.full((16,), -3.0, f32)
            cm1 = jnp.full((16,), -3.0, f32)
            for c in range(_NCH):
                nm = _allmax(bufv[t, pl.ds(c * 16, 16)])
                if c < 16:
                    cm0 = jnp.where(iota == c, nm, cm0)
                else:
                    cm1 = jnp.where(iota == c - 16, nm, cm1)
            idxv[t, pl.ds(96, 16)] = zi
            ssv[t, pl.ds(96, 16)] = zf
            cms += [cm0, cm1]

        def pop1(r16, c0, c1, t16):
            mx = _allmax(jnp.maximum(c0, c1))
            cand = jnp.minimum(jnp.where(c0 == mx, iota, big),
                               jnp.where(c1 == mx, iota + 16, big))
            cb = _allmin(cand)
            cb16 = cb * 16
            chunk = plsc.load_gather(bufv, [t16, cb16 + iota])
            lbest = _allmin(jnp.where(chunk == mx, iota, big))
            gidx = cb16 + lbest
            plsc.store_scatter(ssv, [t16, r16], mx, mask=m0)
            plsc.store_scatter(idxv, [t16, r16], gidx, mask=m0)
            plsc.store_scatter(bufv, [t16, gidx], neg2, mask=m0)
            chunk2 = jnp.where(iota == lbest, -2.0, chunk)
            nm = _allmax(chunk2)
            c0 = jnp.where(iota == cb, nm, c0)
            c1 = jnp.where(iota == cb - 16, nm, c1)
            return c0, c1

        def pop(r, carry):
            r16 = jnp.broadcast_to(r, (16,))
            out = []
            for j in range(_G):
                c0, c1 = pop1(r16, carry[2 * j], carry[2 * j + 1], ts[j][1])
                out += [c0, c1]
            return tuple(out)

        lax.fori_loop(0, _TOPK, pop, tuple(cms))

        for t, tv in ts:
            for c in range(7):
                idx_c = idxv[t, pl.ds(c * 16, 16)]
                sx1[t, pl.ds(c * 16, 16)] = plsc.load_gather(bufv, [tv, idx_c + _SEG])
                sy1[t, pl.ds(c * 16, 16)] = plsc.load_gather(bufv, [tv, idx_c + 2 * _SEG])
                sx2[t, pl.ds(c * 16, 16)] = plsc.load_gather(bufv, [tv, idx_c + 3 * _SEG])
                sy2[t, pl.ds(c * 16, 16)] = plsc.load_gather(bufv, [tv, idx_c + 4 * _SEG])
                supv[t, pl.ds(c * 16, 16)] = jnp.where(iota + c * 16 < _TOPK, 0, 1)

        for blk in range(4):
            c_lo = 2 * blk

            def nms1(i, i16, t, t16):
                xi1 = plsc.load_gather(sx1, [t16, i16])
                xi2 = plsc.load_gather(sy1, [t16, i16])
                xi3 = plsc.load_gather(sx2, [t16, i16])
                xi4 = plsc.load_gather(sy2, [t16, i16])
                ai = (xi3 - xi1) * (xi4 - xi2)
                act = plsc.load_gather(supv, [t16, i16]) == 0
                for c in range(c_lo, 7):
                    b1 = sx1[t, pl.ds(c * 16, 16)]
                    b2 = sy1[t, pl.ds(c * 16, 16)]
                    b3 = sx2[t, pl.ds(c * 16, 16)]
                    b4 = sy2[t, pl.ds(c * 16, 16)]
                    av = (b3 - b1) * (b4 - b2)
                    iw = jnp.maximum(jnp.minimum(b3, xi3) - jnp.maximum(b1, xi1), 0.0)
                    ih = jnp.maximum(jnp.minimum(b4, xi4) - jnp.maximum(b2, xi2), 0.0)
                    inter = iw * ih
                    iou = inter / jnp.maximum(av + ai - inter, 1e-9)
                    sold = supv[t, pl.ds(c * 16, 16)]
                    cond = act & (iou > _THR) & (iota + c * 16 > i)
                    supv[t, pl.ds(c * 16, 16)] = jnp.where(cond, 1, sold)

            def nms(i, carry):
                i16 = jnp.broadcast_to(i, (16,))
                for t, tv in ts:
                    nms1(i, i16, t, tv)
                return carry

            lax.fori_loop(32 * blk, min(32 * blk + 32, _TOPK), nms, 0)

    for t in range(_PPW):
        t16 = jnp.full((16,), t, i32)
        for a in range(_NSEG):
            stf[t, pl.ds(a * 32, 16)] = zf
            stf[t, pl.ds(a * 32 + 16, 16)] = zf
        base = zi
        for c in range(7):
            keep = supv[t, pl.ds(c * 16, 16)] == 0
            ki = keep.astype(i32)
            cum = _cumsum(ki)
            slot = base + cum - ki
            msel = keep & (slot < _KEEP)
            plsc.store_scatter(stf, [t16, slot], ssv[t, pl.ds(c * 16, 16)], mask=msel)
            plsc.store_scatter(stf, [t16, slot + 32], sx1[t, pl.ds(c * 16, 16)], mask=msel)
            plsc.store_scatter(stf, [t16, slot + 64], sy1[t, pl.ds(c * 16, 16)], mask=msel)
            plsc.store_scatter(stf, [t16, slot + 96], sx2[t, pl.ds(c * 16, 16)], mask=msel)
            plsc.store_scatter(stf, [t16, slot + 128], sy2[t, pl.ds(c * 16, 16)], mask=msel)
            base = base + _take(cum, lane15)
        mrow = miv[t, pl.ds(0, 16)]
        name = _allmax(jnp.where(iota == 0, mrow, -1))
        strt = _allmax(jnp.where(iota == 1, mrow, -1))
        endv = _allmax(jnp.where(iota == 2, mrow, -1))
        neg = jnp.full((16,), -1, i32)
        v0 = iota < base
        v1 = (iota + 16) < base
        sti[t, pl.ds(0, 16)] = jnp.where(v0, name, neg)
        sti[t, pl.ds(16, 16)] = jnp.where(v1, name, neg)
        sti[t, pl.ds(32, 16)] = jnp.where(v0, strt, neg)
        sti[t, pl.ds(48, 16)] = jnp.where(v1, strt, neg)
        sti[t, pl.ds(64, 16)] = jnp.where(v0, endv, neg)
        sti[t, pl.ds(80, 16)] = jnp.where(v1, endv, neg)

    pltpu.sync_copy(stf, of_h.at[pl.ds(base_row, _PPW)])
    pltpu.sync_copy(sti, oi_h.at[pl.ds(base_row, _PPW)])


def kernel(pred_logits, pred_boxes, target_sizes, pred_names, mask_infos):
    f32 = jnp.float32
    i32 = jnp.int32
    lg = pred_logits[:, 0, :, 1].reshape(_N, _Q)
    bx = pred_boxes[:, 0].reshape(_N, _Q, 4)
    img_w = jnp.repeat(target_sizes[:, 1], _P)
    img_h = jnp.repeat(target_sizes[:, 0], _P)
    sxf = jnp.broadcast_to(img_w[:, None], (_N, 128))
    syf = jnp.broadcast_to(img_h[:, None], (_N, 128))
    mi = jnp.pad(
        jnp.stack([pred_names.reshape(_N), mask_infos[..., 0].reshape(_N),
                   mask_infos[..., 1].reshape(_N)], axis=-1).astype(i32),
        ((0, 0), (0, 13)))

    buf = pl.pallas_call(
        _prep_body,
        out_shape=jax.ShapeDtypeStruct((_N, _W), f32),
    )(lg, bx[..., 0], bx[..., 1], bx[..., 2], bx[..., 3], sxf, syf)

    mesh = plsc.VectorSubcoreMesh(core_axis_name="c", subcore_axis_name="s",
                                  num_cores=2, num_subcores=16)
    sc = functools.partial(
        pl.kernel, mesh=mesh,
        compiler_params=pltpu.CompilerParams(needs_layout_passes=False),
        out_type=[jax.ShapeDtypeStruct((_N, 32 * _NSEG), f32),
                  jax.ShapeDtypeStruct((_N, 96), i32)],
        scratch_types=[pltpu.VMEM((_PPW, _W), f32), pltpu.VMEM((_PPW, 16), i32),
                       pltpu.VMEM((_PPW, _MS), f32), pltpu.VMEM((_PPW, _MS), i32)]
        + [pltpu.VMEM((_PPW, _MS), f32)] * 4
        + [pltpu.VMEM((_PPW, _MS), i32)]
        + [pltpu.VMEM((_PPW, 32 * _NSEG), f32), pltpu.VMEM((_PPW, 96), i32)],
    )(_sc_body)

    of, oi = sc(buf, mi)
    scores = of[:, 0:_KEEP].reshape(_BS, _P * _KEEP)
    boxes = jnp.stack([of[:, 32:32 + _KEEP], of[:, 64:64 + _KEEP],
                       of[:, 96:96 + _KEEP], of[:, 128:128 + _KEEP]],
                      axis=-1).reshape(_BS, _P * _KEEP, 4)
    names_o = oi[:, 0:_KEEP].reshape(_BS, _P * _KEEP)
    starts_o = oi[:, 32:32 + _KEEP].reshape(_BS, _P * _KEEP)
    ends_o = oi[:, 64:64 + _KEEP].reshape(_BS, _P * _KEEP)
    return scores, boxes, names_o, starts_o, ends_o


# speedup vs baseline: 1.0507x; 1.0068x over previous
import functools

import jax
import jax.numpy as jnp
from jax import lax
from jax.experimental import pallas as pl
from jax.experimental.pallas import tpu as pltpu
from jax.experimental.pallas import tpu_sc as plsc

_BS = 4
_Q = 300
_P = 64
_N = _BS * _P
_NCH = 19
_SEG = 384
_NSEG = 5
_W = _SEG * _NSEG
_TOPK = 100
_MS = 112
_KEEP = 20
_THR = 0.7
_PPW = 8


def _prep_body(lg_ref, cx_ref, cy_ref, w_ref, h_ref, sx_ref, sy_ref, out_ref):
    lanes = lax.broadcasted_iota(jnp.int32, (_N, _SEG), 1)
    pad = ((0, 0), (0, _SEG - _Q))
    x = jnp.pad(lg_ref[...], pad)
    prob = 0.5 * (jnp.tanh(0.5 * x) + 1.0)
    out_ref[:, 0:_SEG] = jnp.where(lanes < _Q, prob, -1.0)
    sx = sx_ref[:, 0:1]
    sy = sy_ref[:, 0:1]
    cx = jnp.pad(cx_ref[...], pad)
    cy = jnp.pad(cy_ref[...], pad)
    bw = jnp.pad(w_ref[...], pad)
    bh = jnp.pad(h_ref[...], pad)
    out_ref[:, _SEG:2 * _SEG] = (cx - 0.5 * bw) * sx
    out_ref[:, 2 * _SEG:3 * _SEG] = (cy - 0.5 * bh) * sy
    out_ref[:, 3 * _SEG:4 * _SEG] = (cx + 0.5 * bw) * sx
    out_ref[:, 4 * _SEG:5 * _SEG] = (cy + 0.5 * bh) * sy


def _sc_body(buf_h, mi_h, of_h, oi_h,
             bufv, miv, ssv, idxv, sx1, sy1, sx2, sy2, supv, stf, sti):
    i32 = jnp.int32
    f32 = jnp.float32
    cid = lax.axis_index("c")
    sid = lax.axis_index("s")
    wid = sid * 2 + cid
    base_row = wid * _PPW
    pltpu.sync_copy(buf_h.at[pl.ds(base_row, _PPW)], bufv)
    pltpu.sync_copy(mi_h.at[pl.ds(base_row, _PPW)], miv)
    iota = lax.broadcasted_iota(i32, (16,), 0)
    zf = jnp.zeros((16,), f32)
    zi = jnp.zeros((16,), i32)
    m0 = iota == 0
    gdn = lax.GatherDimensionNumbers(offset_dims=(), collapsed_slice_dims=(0,),
                                     start_index_map=(0,))

    def _take(x, idx):
        return lax.gather(x, idx[:, None], dimension_numbers=gdn,
                          slice_sizes=(1,),
                          mode=lax.GatherScatterMode.PROMISE_IN_BOUNDS)

    perms = [iota ^ d for d in (1, 2, 4, 8)]

    def _allmax(x):
        for pm in perms:
            x = jnp.maximum(x, _take(x, pm))
        return x

    def _allmin(x):
        for pm in perms:
            x = jnp.minimum(x, _take(x, pm))
        return x

    shift_idx = [(jnp.maximum(iota - d, 0), iota >= d) for d in (1, 2, 4, 8)]

    def _cumsum(x):
        for sidx, smask in shift_idx:
            x = x + jnp.where(smask, _take(x, sidx), 0)
        return x

    lane15 = jnp.full((16,), 15, i32)
    big = jnp.full((16,), 512, i32)
    neg2 = jnp.full((16,), -2.0, f32)

    _G = 8
    for tp in range(_PPW // _G):
        ts = [(_G * tp + j, jnp.full((16,), _G * tp + j, i32)) for j in range(_G)]

        cms = []
        for t, tv in ts:
            cm0 = jnp.full((16,), -3.0, f32)
            cm1 = jnp.full((16,), -3.0, f32)
            for c in range(_NCH):
                nm = _allmax(bufv[t, pl.ds(c * 16, 16)])
                if c < 16:
                    cm0 = jnp.where(iota == c, nm, cm0)
                else:
                    cm1 = jnp.where(iota == c - 16, nm, cm1)
            idxv[t, pl.ds(96, 16)] = zi
            ssv[t, pl.ds(96, 16)] = zf
            cms += [cm0, cm1]

        def pop1(r16, c0, c1, t16):
            mx = _allmax(jnp.maximum(c0, c1))
            cand = jnp.minimum(jnp.where(c0 == mx, iota, big),
                               jnp.where(c1 == mx, iota + 16, big))
            cb = _allmin(cand)
            cb16 = cb * 16
            chunk = plsc.load_gather(bufv, [t16, cb16 + iota])
            lbest = _allmin(jnp.where(chunk == mx, iota, big))
            gidx = cb16 + lbest
            plsc.store_scatter(ssv, [t16, r16], mx, mask=m0)
            plsc.store_scatter(idxv, [t16, r16], gidx, mask=m0)
            plsc.store_scatter(bufv, [t16, gidx], neg2, mask=m0)
            chunk2 = jnp.where(iota == lbest, -2.0, chunk)
            nm = _allmax(chunk2)
            c0 = jnp.where(iota == cb, nm, c0)
            c1 = jnp.where(iota == cb - 16, nm, c1)
            return c0, c1

        def pop(r, carry):
            r16 = jnp.broadcast_to(r, (16,))
            out = []
            for j in range(_G):
                c0, c1 = pop1(r16, carry[2 * j], carry[2 * j + 1], ts[j][1])
                out += [c0, c1]
            return tuple(out)

        lax.fori_loop(0, _TOPK, pop, tuple(cms))

        for t, tv in ts:
            for c in range(7):
                idx_c = idxv[t, pl.ds(c * 16, 16)]
                sx1[t, pl.ds(c * 16, 16)] = plsc.load_gather(bufv, [tv, idx_c + _SEG])
                sy1[t, pl.ds(c * 16, 16)] = plsc.load_gather(bufv, [tv, idx_c + 2 * _SEG])
                sx2[t, pl.ds(c * 16, 16)] = plsc.load_gather(bufv, [tv, idx_c + 3 * _SEG])
                sy2[t, pl.ds(c * 16, 16)] = plsc.load_gather(bufv, [tv, idx_c + 4 * _SEG])
                supv[t, pl.ds(c * 16, 16)] = jnp.where(iota + c * 16 < _TOPK, 0, 1)

        for blk in range(4):
            c_lo = 2 * blk

            def nms1(i, i16, t, t16):
                xi1 = plsc.load_gather(sx1, [t16, i16])
                xi2 = plsc.load_gather(sy1, [t16, i16])
                xi3 = plsc.load_gather(sx2, [t16, i16])
                xi4 = plsc.load_gather(sy2, [t16, i16])
                ai = (xi3 - xi1) * (xi4 - xi2)
                act = plsc.load_gather(supv, [t16, i16]) == 0
                for c in range(c_lo, 7):
                    b1 = sx1[t, pl.ds(c * 16, 16)]
                    b2 = sy1[t, pl.ds(c * 16, 16)]
                    b3 = sx2[t, pl.ds(c * 16, 16)]
                    b4 = sy2[t, pl.ds(c * 16, 16)]
                    av = (b3 - b1) * (b4 - b2)
                    iw = jnp.maximum(jnp.minimum(b3, xi3) - jnp.maximum(b1, xi1), 0.0)
                    ih = jnp.maximum(jnp.minimum(b4, xi4) - jnp.maximum(b2, xi2), 0.0)
                    inter = iw * ih
                    iou = inter / jnp.maximum(av + ai - inter, 1e-9)
                    sold = supv[t, pl.ds(c * 16, 16)]
                    cond = act & (iou > _THR) & (iota + c * 16 > i)
                    supv[t, pl.ds(c * 16, 16)] = jnp.where(cond, 1, sold)

            def nms(i, carry):
                i16 = jnp.broadcast_to(i, (16,))
                for t, tv in ts:
                    nms1(i, i16, t, tv)
                return carry

            lax.fori_loop(32 * blk, min(32 * blk + 32, _TOPK), nms, 0)

    for t in range(_PPW):
        t16 = jnp.full((16,), t, i32)
        for a in range(_NSEG):
            stf[t, pl.ds(a * 32, 16)] = zf
            stf[t, pl.ds(a * 32 + 16, 16)] = zf
        base = zi
        for c in range(7):
            keep = supv[t, pl.ds(c * 16, 16)] == 0
            ki = keep.astype(i32)
            cum = _cumsum(ki)
            slot = base + cum - ki
            msel = keep & (slot < _KEEP)
            plsc.store_scatter(stf, [t16, slot], ssv[t, pl.ds(c * 16, 16)], mask=msel)
            plsc.store_scatter(stf, [t16, slot + 32], sx1[t, pl.ds(c * 16, 16)], mask=msel)
            plsc.store_scatter(stf, [t16, slot + 64], sy1[t, pl.ds(c * 16, 16)], mask=msel)
            plsc.store_scatter(stf, [t16, slot + 96], sx2[t, pl.ds(c * 16, 16)], mask=msel)
            plsc.store_scatter(stf, [t16, slot + 128], sy2[t, pl.ds(c * 16, 16)], mask=msel)
            base = base + _take(cum, lane15)
        mrow = miv[t, pl.ds(0, 16)]
        name = _allmax(jnp.where(iota == 0, mrow, -1))
        strt = _allmax(jnp.where(iota == 1, mrow, -1))
        endv = _allmax(jnp.where(iota == 2, mrow, -1))
        neg = jnp.full((16,), -1, i32)
        v0 = iota < base
        v1 = (iota + 16) < base
        sti[t, pl.ds(0, 16)] = jnp.where(v0, name, neg)
        sti[t, pl.ds(16, 16)] = jnp.where(v1, name, neg)
        sti[t, pl.ds(32, 16)] = jnp.where(v0, strt, neg)
        sti[t, pl.ds(48, 16)] = jnp.where(v1, strt, neg)
        sti[t, pl.ds(64, 16)] = jnp.where(v0, endv, neg)
        sti[t, pl.ds(80, 16)] = jnp.where(v1, endv, neg)

    pltpu.sync_copy(stf, of_h.at[pl.ds(base_row, _PPW)])
    pltpu.sync_copy(sti, oi_h.at[pl.ds(base_row, _PPW)])


def kernel(pred_logits, pred_boxes, target_sizes, pred_names, mask_infos):
    f32 = jnp.float32
    i32 = jnp.int32
    lg = pred_logits[:, 0, :, 1].reshape(_N, _Q)
    bx = pred_boxes[:, 0].reshape(_N, _Q, 4)
    img_w = jnp.repeat(target_sizes[:, 1], _P)
    img_h = jnp.repeat(target_sizes[:, 0], _P)
    sxf = jnp.broadcast_to(img_w[:, None], (_N, 128))
    syf = jnp.broadcast_to(img_h[:, None], (_N, 128))
    mi = jnp.pad(
        jnp.stack([pred_names.reshape(_N), mask_infos[..., 0].reshape(_N),
                   mask_infos[..., 1].reshape(_N)], axis=-1).astype(i32),
        ((0, 0), (0, 13)))

    buf = pl.pallas_call(
        _prep_body,
        out_shape=jax.ShapeDtypeStruct((_N, _W), f32),
    )(lg, bx[..., 0], bx[..., 1], bx[..., 2], bx[..., 3], sxf, syf)

    mesh = plsc.VectorSubcoreMesh(core_axis_name="c", subcore_axis_name="s",
                                  num_cores=2, num_subcores=16)
    sc = functools.partial(
        pl.kernel, mesh=mesh,
        compiler_params=pltpu.CompilerParams(needs_layout_passes=False),
        out_type=[jax.ShapeDtypeStruct((_N, 32 * _NSEG), f32),
                  jax.ShapeDtypeStruct((_N, 96), i32)],
        scratch_types=[pltpu.VMEM((_PPW, _W), f32), pltpu.VMEM((_PPW, 16), i32),
                       pltpu.VMEM((_PPW, _MS), f32), pltpu.VMEM((_PPW, _MS), i32)]
        + [pltpu.VMEM((_PPW, _MS), f32)] * 4
        + [pltpu.VMEM((_PPW, _MS), i32)]
        + [pltpu.VMEM((_PPW, 32 * _NSEG), f32), pltpu.VMEM((_PPW, 96), i32)],
    )(_sc_body)

    of, oi = sc(buf, mi)
    scores = of[:, 0:_KEEP].reshape(_BS, _P * _KEEP)
    boxes = jnp.stack([of[:, 32:32 + _KEEP], of[:, 64:64 + _KEEP],
                       of[:, 96:96 + _KEEP], of[:, 128:128 + _KEEP]],
                      axis=-1).reshape(_BS, _P * _KEEP, 4)
    names_o = oi[:, 0:_KEEP].reshape(_BS, _P * _KEEP)
    starts_o = oi[:, 32:32 + _KEEP].reshape(_BS, _P * _KEEP)
    ends_o = oi[:, 64:64 + _KEEP].reshape(_BS, _P * _KEEP)
    return scores, boxes, names_o, starts_o, ends_o
